# baseline (speedup 1.0000x reference)
import functools

import jax
import jax.numpy as jnp
from jax.experimental import pallas as pl
from jax.experimental.pallas import tpu as pltpu

QUANT_DIM = 8192
EMBED_DIM = 256
TOPK = 15
COMMITMENT_COST = 0.25
TM = 192
NB = 9216 // TM


def _vq_kernel(z_ref, z2_ref, zp_ref, w_ref, w2_ref, coli_ref,
               zq_ref, khot_ref, loss_ref, dbuf):
    i = pl.program_id(0)
    w = w_ref[...]

    @pl.when(i < NB)
    def _():
        mm = jax.lax.dot_general(z_ref[...], w, (((1,), (1,)), ((), ())),
                                 preferred_element_type=jnp.float32)
        dist = (z2_ref[...] - 2.0 * mm) + w2_ref[...]
        bits = jax.lax.bitcast_convert_type(dist, jnp.int32)
        dbuf[i % 2] = (bits << 13) + coli_ref[...]

    @pl.when(i > 0)
    def _():
        ikey = dbuf[(i - 1) % 2]
        key = ikey - ikey[:, 0:1]
        big = jnp.int32(2 ** 31 - 1)
        neg = jnp.int32(-2 ** 31)

        a = jnp.full((TM, 128), big)
        b = a
        c3 = a
        for k in range(QUANT_DIM // 128):
            s = key[:, k * 128:(k + 1) * 128]
            g1 = jnp.maximum(s, a)
            a = jnp.minimum(s, a)
            g2 = jnp.maximum(g1, b)
            b = jnp.minimum(g1, b)
            c3 = jnp.minimum(g2, c3)
        cand = jnp.concatenate([a, b, c3], axis=1)
        t = jnp.min(cand, axis=1, keepdims=True)
        for _ in range(TOPK - 1):
            t = jnp.min(jnp.where(cand > t, cand, big),
                        axis=1, keepdims=True)

        c = jnp.sum((key <= t).astype(jnp.float32), axis=1, keepdims=True)

        def refine_cond(carry):
            _, cc = carry
            return jnp.max(cc) > jnp.float32(TOPK)

        def refine_body(carry):
            tt, cc = carry
            tn = jnp.max(jnp.where(key < tt, key, neg),
                         axis=1, keepdims=True)
            over = cc > jnp.float32(TOPK)
            return jnp.where(over, tn, tt), jnp.where(over, cc - 1.0, cc)

        t, c = jax.lax.while_loop(refine_cond, refine_body, (t, c))
        khot = (key <= t).astype(jnp.float32)
        khot_ref[...] = khot

        zq = jax.lax.dot_general(khot, w, (((1,), (0,)), ((), ())),
                                 preferred_element_type=jnp.float32)
        zp = zp_ref[...]
        zq_ref[...] = zp + (zq - zp)
        diff = zq - zp
        part = jnp.sum(diff * diff).reshape(1, 1)

        @pl.when(i == 1)
        def _():
            loss_ref[...] = part

        @pl.when(i > 1)
        def _():
            loss_ref[...] += part


@jax.jit
def kernel(z_e, W):
    n = z_e.shape[0]
    z2 = jnp.sum(z_e ** 2, axis=1, keepdims=True)
    w2 = jnp.sum(W ** 2, axis=1)[None, :]
    coli = jnp.arange(QUANT_DIM, dtype=jnp.int32)[None, :]
    zq_ste, khot, loss = pl.pallas_call(
        _vq_kernel,
        grid=(NB + 1,),
        in_specs=[
            pl.BlockSpec((TM, EMBED_DIM), lambda i: (jnp.minimum(i, NB - 1), 0)),
            pl.BlockSpec((TM, 1), lambda i: (jnp.minimum(i, NB - 1), 0)),
            pl.BlockSpec((TM, EMBED_DIM), lambda i: (jnp.maximum(i - 1, 0), 0)),
            pl.BlockSpec((QUANT_DIM, EMBED_DIM), lambda i: (0, 0)),
            pl.BlockSpec((1, QUANT_DIM), lambda i: (0, 0)),
            pl.BlockSpec((1, QUANT_DIM), lambda i: (0, 0)),
        ],
        out_specs=[
            pl.BlockSpec((TM, EMBED_DIM), lambda i: (jnp.maximum(i - 1, 0), 0)),
            pl.BlockSpec((TM, QUANT_DIM), lambda i: (jnp.maximum(i - 1, 0), 0)),
            pl.BlockSpec((1, 1), lambda i: (0, 0)),
        ],
        out_shape=[
            jax.ShapeDtypeStruct((n, EMBED_DIM), jnp.float32),
            jax.ShapeDtypeStruct((n, QUANT_DIM), jnp.float32),
            jax.ShapeDtypeStruct((1, 1), jnp.float32),
        ],
        scratch_shapes=[pltpu.VMEM((2, TM, QUANT_DIM), jnp.int32)],
    )(z_e, z2, z_e, W, w2, coli)
    v = loss[0, 0] / jnp.float32(n * EMBED_DIM)
    total = v + jnp.float32(COMMITMENT_COST) * v
    return zq_ste, total, khot


# speedup vs baseline: 1.4082x; 1.4082x over previous
import functools

import jax
import jax.numpy as jnp
from jax.experimental import pallas as pl
from jax.experimental.pallas import tpu as pltpu

QUANT_DIM = 8192
EMBED_DIM = 256
TOPK = 15
COMMITMENT_COST = 0.25
TM = 256
NB = 9216 // TM


def _vq_kernel(z_ref, z2_ref, zp_ref, w_ref, w2_ref, coli_ref,
               zq_ref, khot_ref, loss_ref, dbuf):
    i = pl.program_id(0)
    w = w_ref[...]

    @pl.when(i < NB)
    def _():
        mm = jax.lax.dot_general(z_ref[...], w, (((1,), (1,)), ((), ())),
                                 preferred_element_type=jnp.float32)
        dbuf[i % 2] = (z2_ref[...] - 2.0 * mm) + w2_ref[...]

    @pl.when(i > 0)
    def _():
        dist = dbuf[(i - 1) % 2]

        colf = coli_ref[...]
        big = jnp.float32(jnp.inf)
        neg = -jnp.float32(jnp.inf)
        m0 = jnp.min(functools.reduce(
            jnp.minimum,
            [dist[:, k * 128:(k + 1) * 128]
             for k in range(QUANT_DIM // 128)]),
            axis=1, keepdims=True)
        key = (dist - m0) + colf

        a = jnp.full((TM, 128), big)
        b = a
        c3 = a
        for k in range(QUANT_DIM // 128):
            s = key[:, k * 128:(k + 1) * 128]
            g1 = jnp.maximum(s, a)
            a = jnp.minimum(s, a)
            g2 = jnp.maximum(g1, b)
            b = jnp.minimum(g1, b)
            c3 = jnp.minimum(g2, c3)
        cand = jnp.concatenate([a, b, c3], axis=1)
        t = jnp.min(cand, axis=1, keepdims=True)
        for _ in range(TOPK - 1):
            t = jnp.min(jnp.where(cand > t, cand, big),
                        axis=1, keepdims=True)

        c = jnp.sum((key <= t).astype(jnp.float32), axis=1, keepdims=True)

        def refine_cond(carry):
            _, cc = carry
            return jnp.max(cc) > jnp.float32(TOPK)

        def refine_body(carry):
            tt, cc = carry
            tn = jnp.max(jnp.where(key < tt, key, neg),
                         axis=1, keepdims=True)
            over = cc > jnp.float32(TOPK)
            return jnp.where(over, tn, tt), jnp.where(over, cc - 1.0, cc)

        t, c = jax.lax.while_loop(refine_cond, refine_body, (t, c))
        khot = (key <= t).astype(jnp.float32)
        khot_ref[...] = khot

        zq = jax.lax.dot_general(khot, w, (((1,), (0,)), ((), ())),
                                 preferred_element_type=jnp.float32)
        zp = zp_ref[...]
        zq_ref[...] = zp + (zq - zp)
        diff = zq - zp
        part = jnp.sum(diff * diff).reshape(1, 1)

        @pl.when(i == 1)
        def _():
            loss_ref[...] = part

        @pl.when(i > 1)
        def _():
            loss_ref[...] += part


@jax.jit
def kernel(z_e, W):
    n = z_e.shape[0]
    z2 = jnp.sum(z_e ** 2, axis=1, keepdims=True)
    w2 = jnp.sum(W ** 2, axis=1)[None, :]
    coli = (jnp.arange(QUANT_DIM, dtype=jnp.float32)
            * jnp.float32(2.0 ** -29))[None, :]
    zq_ste, khot, loss = pl.pallas_call(
        _vq_kernel,
        grid=(NB + 1,),
        in_specs=[
            pl.BlockSpec((TM, EMBED_DIM), lambda i: (jnp.minimum(i, NB - 1), 0)),
            pl.BlockSpec((TM, 1), lambda i: (jnp.minimum(i, NB - 1), 0)),
            pl.BlockSpec((TM, EMBED_DIM), lambda i: (jnp.maximum(i - 1, 0), 0)),
            pl.BlockSpec((QUANT_DIM, EMBED_DIM), lambda i: (0, 0)),
            pl.BlockSpec((1, QUANT_DIM), lambda i: (0, 0)),
            pl.BlockSpec((1, QUANT_DIM), lambda i: (0, 0)),
        ],
        out_specs=[
            pl.BlockSpec((TM, EMBED_DIM), lambda i: (jnp.maximum(i - 1, 0), 0)),
            pl.BlockSpec((TM, QUANT_DIM), lambda i: (jnp.maximum(i - 1, 0), 0)),
            pl.BlockSpec((1, 1), lambda i: (0, 0)),
        ],
        out_shape=[
            jax.ShapeDtypeStruct((n, EMBED_DIM), jnp.float32),
            jax.ShapeDtypeStruct((n, QUANT_DIM), jnp.float32),
            jax.ShapeDtypeStruct((1, 1), jnp.float32),
        ],
        scratch_shapes=[pltpu.VMEM((2, TM, QUANT_DIM), jnp.float32)],
    )(z_e, z2, z_e, W, w2, coli)
    v = loss[0, 0] / jnp.float32(n * EMBED_DIM)
    total = v + jnp.float32(COMMITMENT_COST) * v
    return zq_ste, total, khot
